# baseline (speedup 1.0000x reference)
import functools

import jax
import jax.numpy as jnp
import numpy as np
from jax import lax
from jax.experimental import pallas as pl
from jax.experimental.pallas import tpu as pltpu
from jax.experimental.pallas import tpu_sc as plsc

_N = 10000
_E = 320000
_H = 128
_R = 32
_H3 = 3 * _H
_NHALF = 2
_EH = _E // _NHALF

_SCALED_SILU = 1.0 / 0.6
_INV_SQRT_3 = 1.0 / np.sqrt(3.0)
_INV_SQRT_H = 1.0 / np.sqrt(float(_H))

_NC = 2
_NS = 16
_NW = _NC * _NS
_PERW = _EH // _NW
_C = 40
_CHUNKS = _PERW // _C
_RPT = 624
_RTAIL = _N - _NS * _RPT


def _dense_body(x_ref, vec_ref, w1_ref, b1_ref, w2_ref, b2_ref, g_ref, bb_ref,
                out_ref):
    x = x_ref[...]
    mu = jnp.mean(x, axis=-1, keepdims=True)
    var = jnp.mean((x - mu) ** 2, axis=-1, keepdims=True)
    xln = (x - mu) * lax.rsqrt(var + 1e-5) * g_ref[...] + bb_ref[...]
    h = lax.dot_general(xln, w1_ref[...], (((1,), (1,)), ((), ())),
                        preferred_element_type=jnp.float32) + b1_ref[...]
    h = h * jax.nn.sigmoid(h) * _SCALED_SILU
    xh = lax.dot_general(h, w2_ref[...], (((1,), (1,)), ((), ())),
                         preferred_element_type=jnp.float32) + b2_ref[...]
    scale = jnp.concatenate([
        jnp.ones((1, _H), jnp.float32),
        jnp.full((1, _H), _INV_SQRT_3 * _INV_SQRT_H, jnp.float32),
        jnp.full((1, _H), _INV_SQRT_H, jnp.float32),
    ], axis=1)
    lo = lax.bitcast_convert_type(
        (xh * scale).astype(jnp.bfloat16), jnp.uint16).astype(jnp.uint32)
    hi = lax.bitcast_convert_type(
        vec_ref[...].astype(jnp.bfloat16), jnp.uint16).astype(jnp.uint32)
    out_ref[...] = lax.bitcast_convert_type(lo | (hi << 16), jnp.int32)


def _dense(x, vecf, w1, b1, w2, b2, g, bb):
    bn = 2000
    return pl.pallas_call(
        _dense_body,
        grid=(_N // bn,),
        in_specs=[
            pl.BlockSpec((bn, _H), lambda ii: (ii, 0)),
            pl.BlockSpec((bn, _H3), lambda ii: (ii, 0)),
            pl.BlockSpec((_H, _H), lambda ii: (0, 0)),
            pl.BlockSpec((1, _H), lambda ii: (0, 0)),
            pl.BlockSpec((_H3, _H), lambda ii: (0, 0)),
            pl.BlockSpec((1, _H3), lambda ii: (0, 0)),
            pl.BlockSpec((1, _H), lambda ii: (0, 0)),
            pl.BlockSpec((1, _H), lambda ii: (0, 0)),
        ],
        out_specs=pl.BlockSpec((bn, _H3), lambda ii: (ii, 0)),
        out_shape=jax.ShapeDtypeStruct((_N, _H3), jnp.int32),
    )(x, vecf, w1, b1, w2, b2, g, bb)


@functools.cache
def _sc_gather_fn():
    mesh = plsc.VectorSubcoreMesh(core_axis_name="c", subcore_axis_name="s",
                                  num_cores=_NC, num_subcores=_NS)
    T = _CHUNKS

    @functools.partial(
        pl.kernel,
        out_type=jax.ShapeDtypeStruct((_EH, _H3), jnp.int32),
        mesh=mesh,
        scratch_types=[
            pltpu.VMEM((_CHUNKS, _C), jnp.int32),
            pltpu.VMEM((4, _C, _H3), jnp.int32),
            pltpu.SemaphoreType.DMA,
            pltpu.SemaphoreType.DMA,
            pltpu.SemaphoreType.DMA,
            pltpu.SemaphoreType.DMA,
            pltpu.SemaphoreType.DMA,
            pltpu.SemaphoreType.DMA,
            pltpu.SemaphoreType.DMA,
            pltpu.SemaphoreType.DMA,
        ],
    )
    def _sc_gather(tab, jidx3, rows_out, jall, bufs,
                   g0, g1, g2, g3, w0, w1, w2, w3):
        c = lax.axis_index("c")
        s = lax.axis_index("s")
        w = s * _NC + c
        pltpu.sync_copy(jidx3.at[w], jall)
        gsem = (g0, g1, g2, g3)
        wsem = (w0, w1, w2, w3)

        def base(k):
            return w * _PERW + k * _C

        def start(k, b):
            pltpu.async_copy(tab.at[jall.at[k]], bufs.at[b], gsem[b])

        def wait_gather(k, b):
            pltpu.make_async_copy(tab.at[jall.at[k]], bufs.at[b],
                                  gsem[b]).wait()

        def fire_write(k, b):
            pltpu.async_copy(bufs.at[b], rows_out.at[pl.ds(base(k), _C)],
                             wsem[b])

        def wait_write(k, b):
            pltpu.make_async_copy(bufs.at[b], rows_out.at[pl.ds(base(k), _C)],
                                  wsem[b]).wait()

        start(0, 0)
        start(1, 1)
        for t in (0, 1):
            wait_gather(t, t % 4)
            fire_write(t, t % 4)
            start(t + 2, (t + 2) % 4)

        nq = (T - 4) // 4

        def body(q, carry):
            for bb in range(4):
                t = 2 + 4 * q + bb
                sl = (2 + bb) % 4
                wait_gather(t, sl)
                fire_write(t, sl)
                wait_write(t - 2, (sl + 2) % 4)
                start(t + 2, (sl + 2) % 4)
            return carry

        lax.fori_loop(0, nq, body, 0)
        for t in range(2 + 4 * nq, T - 2):
            wait_gather(t, t % 4)
            fire_write(t, t % 4)
            wait_write(t - 2, (t + 2) % 4)
            start(t + 2, (t + 2) % 4)
        for t in (T - 2, T - 1):
            wait_gather(t, t % 4)
            fire_write(t, t % 4)
            wait_write(t - 2, (t - 2) % 4)
        wait_write(T - 2, (T - 2) % 4)
        wait_write(T - 1, (T - 1) % 4)

    return _sc_gather


def _msg_body(rows_ref, rbf_ref, wr_ref, br_ref, ev_ref,
              u1_ref, uv0_ref, uv1_ref, uv2_ref):
    rbfh = lax.dot_general(
        rbf_ref[...], wr_ref[...], (((1,), (1,)), ((), ())),
        preferred_element_type=jnp.float32) + br_ref[...]
    ru = lax.bitcast_convert_type(rows_ref[...], jnp.uint32)
    lo16 = lax.convert_element_type(ru & 0xFFFF, jnp.uint16)
    hi16 = lax.convert_element_type(ru >> 16, jnp.uint16)
    xhj = lax.bitcast_convert_type(lo16, jnp.bfloat16).astype(jnp.float32)
    vecj = lax.bitcast_convert_type(hi16, jnp.bfloat16).astype(jnp.float32)
    m = xhj * rbfh
    m2 = m[:, _H:2 * _H]
    m3 = m[:, 2 * _H:]
    u1_ref[...] = m[:, :_H]
    ev = ev_ref[...]
    for d, ref in enumerate((uv0_ref, uv1_ref, uv2_ref)):
        ref[...] = vecj[:, d * _H:(d + 1) * _H] * m2 + m3 * ev[:, d:d + 1]


def _msg(rows, rbf, wr, br, ev):
    be = 1600
    out_sds = jax.ShapeDtypeStruct((_EH, _H), jnp.float32)
    return pl.pallas_call(
        _msg_body,
        grid=(_EH // be,),
        in_specs=[
            pl.BlockSpec((be, _H3), lambda ii: (ii, 0)),
            pl.BlockSpec((be, _R), lambda ii: (ii, 0)),
            pl.BlockSpec((_H3, _R), lambda ii: (0, 0)),
            pl.BlockSpec((1, _H3), lambda ii: (0, 0)),
            pl.BlockSpec((be, 3), lambda ii: (ii, 0)),
        ],
        out_specs=[pl.BlockSpec((be, _H), lambda ii: (ii, 0))] * 4,
        out_shape=[out_sds] * 4,
    )(rows, rbf, wr, br, ev)


@functools.cache
def _sc_scatter_fn():
    mesh = plsc.VectorSubcoreMesh(core_axis_name="c", subcore_axis_name="s",
                                  num_cores=_NC, num_subcores=_NS)
    T = _CHUNKS

    @functools.partial(
        pl.kernel,
        out_type=jax.ShapeDtypeStruct((4, _NC, _N, _H), jnp.float32),
        mesh=mesh,
        scratch_types=[
            pltpu.VMEM((_CHUNKS, _C), jnp.int32),
            pltpu.VMEM((3, _C, _H), jnp.float32),
            pltpu.VMEM_SHARED((_N, _H), jnp.float32),
            pltpu.SemaphoreType.DMA,
            pltpu.SemaphoreType.DMA,
            pltpu.SemaphoreType.DMA,
        ],
    )
    def _sc_scatter(u1, uv0, uv1, uv2, iidx3, zeros, out,
                    iall, bufs, acc, r0, r1, r2):
        c = lax.axis_index("c")
        s = lax.axis_index("s")
        w = s * _NC + c
        rbase = s * _RPT
        rsem = (r0, r1, r2)

        tail = _NS * _RPT
        pltpu.sync_copy(iidx3.at[w], iall)

        def base(k):
            return w * _PERW + k * _C

        for p, u in enumerate((u1, uv0, uv1, uv2)):
            pltpu.sync_copy(zeros.at[pl.ds(rbase, _RPT)],
                            acc.at[pl.ds(rbase, _RPT)])

            @pl.when(s == _NS - 1)
            def _zero_tail():
                pltpu.sync_copy(zeros.at[pl.ds(tail, _RTAIL)],
                                acc.at[pl.ds(tail, _RTAIL)])

            plsc.subcore_barrier()

            def read(k, b, u=u):
                pltpu.async_copy(u.at[pl.ds(base(k), _C)], bufs.at[b],
                                 rsem[b])

            def wait_read(k, b, u=u):
                pltpu.make_async_copy(u.at[pl.ds(base(k), _C)], bufs.at[b],
                                      rsem[b]).wait()

            def scat(k, b):
                pltpu.sync_copy(bufs.at[b], acc.at[iall.at[k]], add=True)

            for t in range(3):
                read(t, t)

            nq = (T - 3) // 3

            def body(q, carry):
                for bb in range(3):
                    t = 3 * q + bb
                    wait_read(t, bb)
                    scat(t, bb)
                    read(t + 3, bb)
                return carry

            lax.fori_loop(0, nq, body, 0)
            for t in range(3 * nq, T):
                sl = t % 3
                wait_read(t, sl)
                scat(t, sl)
                if t + 3 < T:
                    read(t + 3, sl)
            plsc.subcore_barrier()
            pltpu.sync_copy(acc.at[pl.ds(rbase, _RPT)],
                            out.at[p, c].at[pl.ds(rbase, _RPT)])

            @pl.when(s == _NS - 1)
            def _dump_tail():
                pltpu.sync_copy(acc.at[pl.ds(tail, _RTAIL)],
                                out.at[p, c].at[pl.ds(tail, _RTAIL)])

            plsc.subcore_barrier()

    return _sc_scatter


def kernel(x, vec, edge_index, edge_rbf, edge_vector, W1, b1, W2, b2, Wr, br,
           ln_g, ln_b):
    tab = _dense(x, vec.reshape(_N, _H3), W1, b1.reshape(1, -1), W2,
                 b2.reshape(1, -1), ln_g.reshape(1, -1), ln_b.reshape(1, -1))
    zeros = jnp.zeros((_N, _H), jnp.float32)
    brr = br.reshape(1, -1)
    gather = _sc_gather_fn()
    scatter = _sc_scatter_fn()
    parts = []
    for h in range(_NHALF):
        sl = slice(h * _EH, (h + 1) * _EH)
        jh = edge_index[0, sl].reshape(_NW, _CHUNKS, _C)
        ih = edge_index[1, sl].reshape(_NW, _CHUNKS, _C)
        rows = gather(tab, jh)
        u1, uv0, uv1, uv2 = _msg(rows, edge_rbf[sl], Wr, brr,
                                 edge_vector[sl])
        parts.append(scatter(u1, uv0, uv1, uv2, ih, zeros))
    pa, pb = parts
    tot = pa[:, 0] + pa[:, 1] + pb[:, 0] + pb[:, 1]
    dx = tot[0]
    dvec = jnp.stack([tot[1], tot[2], tot[3]], axis=1)
    return dx, dvec


# speedup vs baseline: 1.0127x; 1.0127x over previous
import functools

import jax
import jax.numpy as jnp
import numpy as np
from jax import lax
from jax.experimental import pallas as pl
from jax.experimental.pallas import tpu as pltpu
from jax.experimental.pallas import tpu_sc as plsc

_N = 10000
_E = 320000
_H = 128
_R = 32
_H3 = 3 * _H
_NHALF = 1
_EH = _E // _NHALF

_SCALED_SILU = 1.0 / 0.6
_INV_SQRT_3 = 1.0 / np.sqrt(3.0)
_INV_SQRT_H = 1.0 / np.sqrt(float(_H))

_NC = 2
_NS = 16
_NW = _NC * _NS
_PERW = _EH // _NW
_C = 40
_CHUNKS = _PERW // _C
_CS = 80
_SCHUNKS = _PERW // _CS
_RPT = 624
_RTAIL = _N - _NS * _RPT


def _dense_body(x_ref, vec_ref, w1_ref, b1_ref, w2_ref, b2_ref, g_ref, bb_ref,
                out_ref):
    x = x_ref[...]
    mu = jnp.mean(x, axis=-1, keepdims=True)
    var = jnp.mean((x - mu) ** 2, axis=-1, keepdims=True)
    xln = (x - mu) * lax.rsqrt(var + 1e-5) * g_ref[...] + bb_ref[...]
    h = lax.dot_general(xln, w1_ref[...], (((1,), (1,)), ((), ())),
                        preferred_element_type=jnp.float32) + b1_ref[...]
    h = h * jax.nn.sigmoid(h) * _SCALED_SILU
    xh = lax.dot_general(h, w2_ref[...], (((1,), (1,)), ((), ())),
                         preferred_element_type=jnp.float32) + b2_ref[...]
    scale = jnp.concatenate([
        jnp.ones((1, _H), jnp.float32),
        jnp.full((1, _H), _INV_SQRT_3 * _INV_SQRT_H, jnp.float32),
        jnp.full((1, _H), _INV_SQRT_H, jnp.float32),
    ], axis=1)
    lo = lax.bitcast_convert_type(
        (xh * scale).astype(jnp.bfloat16), jnp.uint16).astype(jnp.uint32)
    hi = lax.bitcast_convert_type(
        vec_ref[...].astype(jnp.bfloat16), jnp.uint16).astype(jnp.uint32)
    out_ref[...] = lax.bitcast_convert_type(lo | (hi << 16), jnp.int32)


def _dense(x, vecf, w1, b1, w2, b2, g, bb):
    bn = 2000
    return pl.pallas_call(
        _dense_body,
        grid=(_N // bn,),
        in_specs=[
            pl.BlockSpec((bn, _H), lambda ii: (ii, 0)),
            pl.BlockSpec((bn, _H3), lambda ii: (ii, 0)),
            pl.BlockSpec((_H, _H), lambda ii: (0, 0)),
            pl.BlockSpec((1, _H), lambda ii: (0, 0)),
            pl.BlockSpec((_H3, _H), lambda ii: (0, 0)),
            pl.BlockSpec((1, _H3), lambda ii: (0, 0)),
            pl.BlockSpec((1, _H), lambda ii: (0, 0)),
            pl.BlockSpec((1, _H), lambda ii: (0, 0)),
        ],
        out_specs=pl.BlockSpec((bn, _H3), lambda ii: (ii, 0)),
        out_shape=jax.ShapeDtypeStruct((_N, _H3), jnp.int32),
    )(x, vecf, w1, b1, w2, b2, g, bb)


@functools.cache
def _sc_gather_fn():
    mesh = plsc.VectorSubcoreMesh(core_axis_name="c", subcore_axis_name="s",
                                  num_cores=_NC, num_subcores=_NS)
    T = _CHUNKS

    @functools.partial(
        pl.kernel,
        out_type=jax.ShapeDtypeStruct((_EH, _H3), jnp.int32),
        mesh=mesh,
        scratch_types=[
            pltpu.VMEM((_CHUNKS, _C), jnp.int32),
            pltpu.VMEM((4, _C, _H3), jnp.int32),
            pltpu.SemaphoreType.DMA,
            pltpu.SemaphoreType.DMA,
            pltpu.SemaphoreType.DMA,
            pltpu.SemaphoreType.DMA,
            pltpu.SemaphoreType.DMA,
            pltpu.SemaphoreType.DMA,
            pltpu.SemaphoreType.DMA,
            pltpu.SemaphoreType.DMA,
        ],
    )
    def _sc_gather(tab, jidx3, rows_out, jall, bufs,
                   g0, g1, g2, g3, w0, w1, w2, w3):
        c = lax.axis_index("c")
        s = lax.axis_index("s")
        w = s * _NC + c
        pltpu.sync_copy(jidx3.at[w], jall)
        gsem = (g0, g1, g2, g3)
        wsem = (w0, w1, w2, w3)

        def base(k):
            return w * _PERW + k * _C

        def start(k, b):
            pltpu.async_copy(tab.at[jall.at[k]], bufs.at[b], gsem[b])

        def wait_gather(k, b):
            pltpu.make_async_copy(tab.at[jall.at[k]], bufs.at[b],
                                  gsem[b]).wait()

        def fire_write(k, b):
            pltpu.async_copy(bufs.at[b], rows_out.at[pl.ds(base(k), _C)],
                             wsem[b])

        def wait_write(k, b):
            pltpu.make_async_copy(bufs.at[b], rows_out.at[pl.ds(base(k), _C)],
                                  wsem[b]).wait()

        start(0, 0)
        start(1, 1)
        for t in (0, 1):
            wait_gather(t, t % 4)
            fire_write(t, t % 4)
            start(t + 2, (t + 2) % 4)

        nq = (T - 4) // 4

        def body(q, carry):
            for bb in range(4):
                t = 2 + 4 * q + bb
                sl = (2 + bb) % 4
                wait_gather(t, sl)
                fire_write(t, sl)
                wait_write(t - 2, (sl + 2) % 4)
                start(t + 2, (sl + 2) % 4)
            return carry

        lax.fori_loop(0, nq, body, 0)
        for t in range(2 + 4 * nq, T - 2):
            wait_gather(t, t % 4)
            fire_write(t, t % 4)
            wait_write(t - 2, (t + 2) % 4)
            start(t + 2, (t + 2) % 4)
        for t in (T - 2, T - 1):
            wait_gather(t, t % 4)
            fire_write(t, t % 4)
            wait_write(t - 2, (t - 2) % 4)
        wait_write(T - 2, (T - 2) % 4)
        wait_write(T - 1, (T - 1) % 4)

    return _sc_gather


def _msg_body(rows_ref, rbf_ref, wr_ref, br_ref, ev_ref,
              u1_ref, uv0_ref, uv1_ref, uv2_ref):
    rbfh = lax.dot_general(
        rbf_ref[...], wr_ref[...], (((1,), (1,)), ((), ())),
        preferred_element_type=jnp.float32) + br_ref[...]
    ru = lax.bitcast_convert_type(rows_ref[...], jnp.uint32)
    lo16 = lax.convert_element_type(ru & 0xFFFF, jnp.uint16)
    hi16 = lax.convert_element_type(ru >> 16, jnp.uint16)
    xhj = lax.bitcast_convert_type(lo16, jnp.bfloat16).astype(jnp.float32)
    vecj = lax.bitcast_convert_type(hi16, jnp.bfloat16).astype(jnp.float32)
    m = xhj * rbfh
    m2 = m[:, _H:2 * _H]
    m3 = m[:, 2 * _H:]
    u1_ref[...] = m[:, :_H]
    ev = ev_ref[...]
    for d, ref in enumerate((uv0_ref, uv1_ref, uv2_ref)):
        ref[...] = vecj[:, d * _H:(d + 1) * _H] * m2 + m3 * ev[:, d:d + 1]


def _msg(rows, rbf, wr, br, ev):
    be = 1600
    out_sds = jax.ShapeDtypeStruct((_EH, _H), jnp.float32)
    return pl.pallas_call(
        _msg_body,
        grid=(_EH // be,),
        in_specs=[
            pl.BlockSpec((be, _H3), lambda ii: (ii, 0)),
            pl.BlockSpec((be, _R), lambda ii: (ii, 0)),
            pl.BlockSpec((_H3, _R), lambda ii: (0, 0)),
            pl.BlockSpec((1, _H3), lambda ii: (0, 0)),
            pl.BlockSpec((be, 3), lambda ii: (ii, 0)),
        ],
        out_specs=[pl.BlockSpec((be, _H), lambda ii: (ii, 0))] * 4,
        out_shape=[out_sds] * 4,
    )(rows, rbf, wr, br, ev)


@functools.cache
def _sc_scatter_fn():
    mesh = plsc.VectorSubcoreMesh(core_axis_name="c", subcore_axis_name="s",
                                  num_cores=_NC, num_subcores=_NS)
    T = _SCHUNKS

    @functools.partial(
        pl.kernel,
        out_type=jax.ShapeDtypeStruct((4, _NC, _N, _H), jnp.float32),
        mesh=mesh,
        scratch_types=[
            pltpu.VMEM((_SCHUNKS, _CS), jnp.int32),
            pltpu.VMEM((3, _CS, _H), jnp.float32),
            pltpu.VMEM_SHARED((_N, _H), jnp.float32),
            pltpu.SemaphoreType.DMA,
            pltpu.SemaphoreType.DMA,
            pltpu.SemaphoreType.DMA,
        ],
    )
    def _sc_scatter(u1, uv0, uv1, uv2, iidx3, zeros, out,
                    iall, bufs, acc, r0, r1, r2):
        c = lax.axis_index("c")
        s = lax.axis_index("s")
        w = s * _NC + c
        rbase = s * _RPT
        rsem = (r0, r1, r2)

        tail = _NS * _RPT
        pltpu.sync_copy(iidx3.at[w], iall)

        def base(k):
            return w * _PERW + k * _CS

        for p, u in enumerate((u1, uv0, uv1, uv2)):
            pltpu.sync_copy(zeros.at[pl.ds(rbase, _RPT)],
                            acc.at[pl.ds(rbase, _RPT)])

            @pl.when(s == _NS - 1)
            def _zero_tail():
                pltpu.sync_copy(zeros.at[pl.ds(tail, _RTAIL)],
                                acc.at[pl.ds(tail, _RTAIL)])

            plsc.subcore_barrier()

            def read(k, b, u=u):
                pltpu.async_copy(u.at[pl.ds(base(k), _CS)], bufs.at[b],
                                 rsem[b])

            def wait_read(k, b, u=u):
                pltpu.make_async_copy(u.at[pl.ds(base(k), _CS)], bufs.at[b],
                                      rsem[b]).wait()

            def scat(k, b):
                pltpu.sync_copy(bufs.at[b], acc.at[iall.at[k]], add=True)

            for t in range(3):
                read(t, t)

            nq = (T - 3) // 3

            def body(q, carry):
                for bb in range(3):
                    t = 3 * q + bb
                    wait_read(t, bb)
                    scat(t, bb)
                    read(t + 3, bb)
                return carry

            lax.fori_loop(0, nq, body, 0)
            for t in range(3 * nq, T):
                sl = t % 3
                wait_read(t, sl)
                scat(t, sl)
                if t + 3 < T:
                    read(t + 3, sl)
            plsc.subcore_barrier()
            pltpu.sync_copy(acc.at[pl.ds(rbase, _RPT)],
                            out.at[p, c].at[pl.ds(rbase, _RPT)])

            @pl.when(s == _NS - 1)
            def _dump_tail():
                pltpu.sync_copy(acc.at[pl.ds(tail, _RTAIL)],
                                out.at[p, c].at[pl.ds(tail, _RTAIL)])

            plsc.subcore_barrier()

    return _sc_scatter


def kernel(x, vec, edge_index, edge_rbf, edge_vector, W1, b1, W2, b2, Wr, br,
           ln_g, ln_b):
    tab = _dense(x, vec.reshape(_N, _H3), W1, b1.reshape(1, -1), W2,
                 b2.reshape(1, -1), ln_g.reshape(1, -1), ln_b.reshape(1, -1))
    zeros = jnp.zeros((_N, _H), jnp.float32)
    brr = br.reshape(1, -1)
    gather = _sc_gather_fn()
    scatter = _sc_scatter_fn()
    parts = []
    for h in range(_NHALF):
        sl = slice(h * _EH, (h + 1) * _EH)
        jh = edge_index[0, sl].reshape(_NW, _CHUNKS, _C)
        ih = edge_index[1, sl].reshape(_NW, _SCHUNKS, _CS)
        rows = gather(tab, jh)
        u1, uv0, uv1, uv2 = _msg(rows, edge_rbf[sl], Wr, brr,
                                 edge_vector[sl])
        parts.append(scatter(u1, uv0, uv1, uv2, ih, zeros))
    tot = sum(p[:, 0] + p[:, 1] for p in parts)
    dx = tot[0]
    dvec = jnp.stack([tot[1], tot[2], tot[3]], axis=1)
    return dx, dvec


# speedup vs baseline: 1.0132x; 1.0005x over previous
import functools

import jax
import jax.numpy as jnp
import numpy as np
from jax import lax
from jax.experimental import pallas as pl
from jax.experimental.pallas import tpu as pltpu
from jax.experimental.pallas import tpu_sc as plsc

_N = 10000
_E = 320000
_H = 128
_R = 32
_H3 = 3 * _H
_NHALF = 1
_EH = _E // _NHALF

_SCALED_SILU = 1.0 / 0.6
_INV_SQRT_3 = 1.0 / np.sqrt(3.0)
_INV_SQRT_H = 1.0 / np.sqrt(float(_H))

_NC = 2
_NS = 16
_NW = _NC * _NS
_PERW = _EH // _NW
_C = 40
_CHUNKS = _PERW // _C
_CS = 80
_SCHUNKS = _PERW // _CS
_RPT = 624
_RTAIL = _N - _NS * _RPT


def _dense_body(x_ref, vec_ref, w1_ref, b1_ref, w2_ref, b2_ref, g_ref, bb_ref,
                out_ref):
    x = x_ref[...]
    mu = jnp.mean(x, axis=-1, keepdims=True)
    var = jnp.mean((x - mu) ** 2, axis=-1, keepdims=True)
    xln = (x - mu) * lax.rsqrt(var + 1e-5) * g_ref[...] + bb_ref[...]
    h = lax.dot_general(xln, w1_ref[...], (((1,), (1,)), ((), ())),
                        preferred_element_type=jnp.float32) + b1_ref[...]
    h = h * jax.nn.sigmoid(h) * _SCALED_SILU
    xh = lax.dot_general(h, w2_ref[...], (((1,), (1,)), ((), ())),
                         preferred_element_type=jnp.float32) + b2_ref[...]
    scale = jnp.concatenate([
        jnp.ones((1, _H), jnp.float32),
        jnp.full((1, _H), _INV_SQRT_3 * _INV_SQRT_H, jnp.float32),
        jnp.full((1, _H), _INV_SQRT_H, jnp.float32),
    ], axis=1)
    lo = lax.bitcast_convert_type(
        (xh * scale).astype(jnp.bfloat16), jnp.uint16).astype(jnp.uint32)
    hi = lax.bitcast_convert_type(
        vec_ref[...].astype(jnp.bfloat16), jnp.uint16).astype(jnp.uint32)
    out_ref[...] = lax.bitcast_convert_type(lo | (hi << 16), jnp.int32)


def _dense(x, vecf, w1, b1, w2, b2, g, bb):
    bn = 2000
    return pl.pallas_call(
        _dense_body,
        grid=(_N // bn,),
        in_specs=[
            pl.BlockSpec((bn, _H), lambda ii: (ii, 0)),
            pl.BlockSpec((bn, _H3), lambda ii: (ii, 0)),
            pl.BlockSpec((_H, _H), lambda ii: (0, 0)),
            pl.BlockSpec((1, _H), lambda ii: (0, 0)),
            pl.BlockSpec((_H3, _H), lambda ii: (0, 0)),
            pl.BlockSpec((1, _H3), lambda ii: (0, 0)),
            pl.BlockSpec((1, _H), lambda ii: (0, 0)),
            pl.BlockSpec((1, _H), lambda ii: (0, 0)),
        ],
        out_specs=pl.BlockSpec((bn, _H3), lambda ii: (ii, 0)),
        out_shape=jax.ShapeDtypeStruct((_N, _H3), jnp.int32),
    )(x, vecf, w1, b1, w2, b2, g, bb)


@functools.cache
def _sc_gather_fn():
    mesh = plsc.VectorSubcoreMesh(core_axis_name="c", subcore_axis_name="s",
                                  num_cores=_NC, num_subcores=_NS)
    T = _CHUNKS

    @functools.partial(
        pl.kernel,
        out_type=jax.ShapeDtypeStruct((_EH, _H3), jnp.int32),
        mesh=mesh,
        scratch_types=[
            pltpu.VMEM((_CHUNKS, _C), jnp.int32),
            pltpu.VMEM((4, _C, _H3), jnp.int32),
            pltpu.SemaphoreType.DMA,
            pltpu.SemaphoreType.DMA,
            pltpu.SemaphoreType.DMA,
            pltpu.SemaphoreType.DMA,
            pltpu.SemaphoreType.DMA,
            pltpu.SemaphoreType.DMA,
            pltpu.SemaphoreType.DMA,
            pltpu.SemaphoreType.DMA,
        ],
    )
    def _sc_gather(tab, jidx3, rows_out, jall, bufs,
                   g0, g1, g2, g3, w0, w1, w2, w3):
        c = lax.axis_index("c")
        s = lax.axis_index("s")
        w = s * _NC + c
        pltpu.sync_copy(jidx3.at[w], jall)
        gsem = (g0, g1, g2, g3)
        wsem = (w0, w1, w2, w3)

        def base(k):
            return w * _PERW + k * _C

        def start(k, b):
            pltpu.async_copy(tab.at[jall.at[k]], bufs.at[b], gsem[b])

        def wait_gather(k, b):
            pltpu.make_async_copy(tab.at[jall.at[k]], bufs.at[b],
                                  gsem[b]).wait()

        def fire_write(k, b):
            pltpu.async_copy(bufs.at[b], rows_out.at[pl.ds(base(k), _C)],
                             wsem[b])

        def wait_write(k, b):
            pltpu.make_async_copy(bufs.at[b], rows_out.at[pl.ds(base(k), _C)],
                                  wsem[b]).wait()

        D, L = 4, 2
        for t in range(L):
            start(t, t % D)
        for t in range(L):
            wait_gather(t, t % D)
            fire_write(t, t % D)
            start(t + L, (t + L) % D)

        nq = (T - 2 * L) // D

        def body(q, carry):
            for bb in range(D):
                t = L + D * q + bb
                sl = (L + bb) % D
                wait_gather(t, sl)
                fire_write(t, sl)
                wait_write(t - L, (sl + 2) % D)
                start(t + L, (sl + 2) % D)
            return carry

        lax.fori_loop(0, nq, body, 0)
        for t in range(L + D * nq, T - L):
            wait_gather(t, t % D)
            fire_write(t, t % D)
            wait_write(t - L, (t + 2) % D)
            start(t + L, (t + 2) % D)
        for t in range(T - L, T):
            wait_gather(t, t % D)
            fire_write(t, t % D)
            wait_write(t - L, (t - L) % D)
        for t in range(T - L, T):
            wait_write(t, t % D)

    return _sc_gather


def _msg_body(rows_ref, rbf_ref, wr_ref, br_ref, ev_ref,
              u1_ref, uv0_ref, uv1_ref, uv2_ref):
    rbfh = lax.dot_general(
        rbf_ref[...], wr_ref[...], (((1,), (1,)), ((), ())),
        preferred_element_type=jnp.float32) + br_ref[...]
    ru = lax.bitcast_convert_type(rows_ref[...], jnp.uint32)
    lo16 = lax.convert_element_type(ru & 0xFFFF, jnp.uint16)
    hi16 = lax.convert_element_type(ru >> 16, jnp.uint16)
    xhj = lax.bitcast_convert_type(lo16, jnp.bfloat16).astype(jnp.float32)
    vecj = lax.bitcast_convert_type(hi16, jnp.bfloat16).astype(jnp.float32)
    m = xhj * rbfh
    m2 = m[:, _H:2 * _H]
    m3 = m[:, 2 * _H:]
    u1_ref[...] = m[:, :_H]
    ev = ev_ref[...]
    for d, ref in enumerate((uv0_ref, uv1_ref, uv2_ref)):
        ref[...] = vecj[:, d * _H:(d + 1) * _H] * m2 + m3 * ev[:, d:d + 1]


def _msg(rows, rbf, wr, br, ev):
    be = 1600
    out_sds = jax.ShapeDtypeStruct((_EH, _H), jnp.float32)
    return pl.pallas_call(
        _msg_body,
        grid=(_EH // be,),
        in_specs=[
            pl.BlockSpec((be, _H3), lambda ii: (ii, 0)),
            pl.BlockSpec((be, _R), lambda ii: (ii, 0)),
            pl.BlockSpec((_H3, _R), lambda ii: (0, 0)),
            pl.BlockSpec((1, _H3), lambda ii: (0, 0)),
            pl.BlockSpec((be, 3), lambda ii: (ii, 0)),
        ],
        out_specs=[pl.BlockSpec((be, _H), lambda ii: (ii, 0))] * 4,
        out_shape=[out_sds] * 4,
    )(rows, rbf, wr, br, ev)


@functools.cache
def _sc_scatter_fn():
    mesh = plsc.VectorSubcoreMesh(core_axis_name="c", subcore_axis_name="s",
                                  num_cores=_NC, num_subcores=_NS)
    T = _SCHUNKS

    @functools.partial(
        pl.kernel,
        out_type=jax.ShapeDtypeStruct((4, _NC, _N, _H), jnp.float32),
        mesh=mesh,
        scratch_types=[
            pltpu.VMEM((_SCHUNKS, _CS), jnp.int32),
            pltpu.VMEM((3, _CS, _H), jnp.float32),
            pltpu.VMEM_SHARED((_N, _H), jnp.float32),
            pltpu.SemaphoreType.DMA,
            pltpu.SemaphoreType.DMA,
            pltpu.SemaphoreType.DMA,
        ],
    )
    def _sc_scatter(u1, uv0, uv1, uv2, iidx3, zeros, out,
                    iall, bufs, acc, r0, r1, r2):
        c = lax.axis_index("c")
        s = lax.axis_index("s")
        w = s * _NC + c
        rbase = s * _RPT
        rsem = (r0, r1, r2)

        tail = _NS * _RPT
        pltpu.sync_copy(iidx3.at[w], iall)

        def base(k):
            return w * _PERW + k * _CS

        for p, u in enumerate((u1, uv0, uv1, uv2)):
            pltpu.sync_copy(zeros.at[pl.ds(rbase, _RPT)],
                            acc.at[pl.ds(rbase, _RPT)])

            @pl.when(s == _NS - 1)
            def _zero_tail():
                pltpu.sync_copy(zeros.at[pl.ds(tail, _RTAIL)],
                                acc.at[pl.ds(tail, _RTAIL)])

            plsc.subcore_barrier()

            def read(k, b, u=u):
                pltpu.async_copy(u.at[pl.ds(base(k), _CS)], bufs.at[b],
                                 rsem[b])

            def wait_read(k, b, u=u):
                pltpu.make_async_copy(u.at[pl.ds(base(k), _CS)], bufs.at[b],
                                      rsem[b]).wait()

            def scat(k, b):
                pltpu.sync_copy(bufs.at[b], acc.at[iall.at[k]], add=True)

            for t in range(3):
                read(t, t)

            nq = (T - 3) // 3

            def body(q, carry):
                for bb in range(3):
                    t = 3 * q + bb
                    wait_read(t, bb)
                    scat(t, bb)
                    read(t + 3, bb)
                return carry

            lax.fori_loop(0, nq, body, 0)
            for t in range(3 * nq, T):
                sl = t % 3
                wait_read(t, sl)
                scat(t, sl)
                if t + 3 < T:
                    read(t + 3, sl)
            plsc.subcore_barrier()
            pltpu.sync_copy(acc.at[pl.ds(rbase, _RPT)],
                            out.at[p, c].at[pl.ds(rbase, _RPT)])

            @pl.when(s == _NS - 1)
            def _dump_tail():
                pltpu.sync_copy(acc.at[pl.ds(tail, _RTAIL)],
                                out.at[p, c].at[pl.ds(tail, _RTAIL)])

            plsc.subcore_barrier()

    return _sc_scatter


def kernel(x, vec, edge_index, edge_rbf, edge_vector, W1, b1, W2, b2, Wr, br,
           ln_g, ln_b):
    tab = _dense(x, vec.reshape(_N, _H3), W1, b1.reshape(1, -1), W2,
                 b2.reshape(1, -1), ln_g.reshape(1, -1), ln_b.reshape(1, -1))
    zeros = jnp.zeros((_N, _H), jnp.float32)
    brr = br.reshape(1, -1)
    gather = _sc_gather_fn()
    scatter = _sc_scatter_fn()
    parts = []
    for h in range(_NHALF):
        sl = slice(h * _EH, (h + 1) * _EH)
        jh = edge_index[0, sl].reshape(_NW, _CHUNKS, _C)
        ih = edge_index[1, sl].reshape(_NW, _SCHUNKS, _CS)
        rows = gather(tab, jh)
        u1, uv0, uv1, uv2 = _msg(rows, edge_rbf[sl], Wr, brr,
                                 edge_vector[sl])
        parts.append(scatter(u1, uv0, uv1, uv2, ih, zeros))
    tot = sum(p[:, 0] + p[:, 1] for p in parts)
    dx = tot[0]
    dvec = jnp.stack([tot[1], tot[2], tot[3]], axis=1)
    return dx, dvec
